# baseline (device time: 287399 ns/iter reference)
import jax
import jax.numpy as jnp
from jax import lax
from jax.experimental import pallas as pl
from jax.experimental.pallas import tpu as pltpu

M = 4096
D = 4096
N_RING = 8
CH = M // N_RING
HCH = CH // 2
XCOLS = 1408
RCOLS = D - XCOLS


def _ring_yz(pos):
    y = jnp.where(pos < 4, 0, 1)
    z = jnp.where(pos < 4, pos, 7 - pos)
    return y, z


def kernel(dy, W):
    my_y = lax.axis_index("y")
    my_z = lax.axis_index("z")
    r = jnp.where(my_y == 0, my_z, 7 - my_z)

    dy_c = lax.dynamic_slice_in_dim(dy, r * CH, CH, axis=0)
    partial = lax.dot_general(
        dy_c,
        W,
        dimension_numbers=(((1,), (1,)), ((), ())),
        precision=lax.Precision.DEFAULT,
    ).astype(jnp.bfloat16)
    return _allreduce(partial).astype(jnp.float32)


def _allreduce(partial):
    def body(
        p_ref, out_ref, xrecv,
        xsa_sem, xra_sem, xsb_sem, xrb_sem,
        fs_sems, fr_sems, bs_sems, br_sems, xfs_sems, xfr_sems,
    ):
        my_x = lax.axis_index("x")
        my_y = lax.axis_index("y")
        my_z = lax.axis_index("z")
        r = jnp.where(my_y == 0, my_z, 7 - my_z)

        ry, rz = _ring_yz((r + 1) % N_RING)
        ly, lz = _ring_yz((r + 7) % N_RING)
        partner = (1 - my_x, my_y, my_z)
        right = (my_x, ry, rz)
        left = (my_x, ly, lz)

        ring_lo = jnp.where(my_x == 0, 0, XCOLS)
        xsend_lo = jnp.where(my_x == 0, 0, RCOLS)
        mylack_lo = jnp.where(my_x == 0, RCOLS, 0)
        partner_ring_lo = jnp.where(my_x == 0, XCOLS, 0)

        barrier_sem = pltpu.get_barrier_semaphore()
        for nbr in (partner, right, left):
            pl.semaphore_signal(
                barrier_sem, inc=1, device_id=nbr,
                device_id_type=pl.DeviceIdType.MESH,
            )
        pl.semaphore_wait(barrier_sem, 3)

        xa = pltpu.make_async_remote_copy(
            src_ref=p_ref.at[:, pl.ds(partner_ring_lo, RCOLS)],
            dst_ref=xrecv.at[:, pl.ds(partner_ring_lo, RCOLS)],
            send_sem=xsa_sem, recv_sem=xra_sem,
            device_id=partner, device_id_type=pl.DeviceIdType.MESH,
        )
        xb = pltpu.make_async_remote_copy(
            src_ref=p_ref.at[:, pl.ds(xsend_lo, XCOLS)],
            dst_ref=xrecv.at[:, pl.ds(xsend_lo, XCOLS)],
            send_sem=xsb_sem, recv_sem=xrb_sem,
            device_id=partner, device_id_type=pl.DeviceIdType.MESH,
        )
        xa.start()
        xb.start()

        def ring_copy(origin, rows_lo, rows, target, send_sem, recv_sem):
            ref = out_ref.at[pl.ds(origin * CH + rows_lo, rows),
                             pl.ds(ring_lo, RCOLS)]
            return pltpu.make_async_remote_copy(
                src_ref=ref, dst_ref=ref, send_sem=send_sem, recv_sem=recv_sem,
                device_id=target, device_id_type=pl.DeviceIdType.MESH,
            )

        def x_copy(origin, k):
            ref = out_ref.at[pl.ds(origin * CH, CH), pl.ds(xsend_lo, XCOLS)]
            return pltpu.make_async_remote_copy(
                src_ref=ref, dst_ref=ref,
                send_sem=xfs_sems.at[k], recv_sem=xfr_sems.at[k],
                device_id=partner, device_id_type=pl.DeviceIdType.MESH,
            )

        def make_step(s):
            if s < 3:
                fwd = ring_copy((r - s + N_RING) % N_RING, 0, CH, right,
                                fs_sems.at[s], fr_sems.at[s])
                bwd = ring_copy((r + s) % N_RING, 0, CH, left,
                                bs_sems.at[s], br_sems.at[s])
            else:
                fwd = ring_copy((r - 3 + N_RING) % N_RING, 0, HCH, right,
                                fs_sems.at[s], fr_sems.at[s])
                bwd = ring_copy((r + 3) % N_RING, HCH, HCH, left,
                                bs_sems.at[s], br_sems.at[s])
            fwd.start()
            bwd.start()
            return fwd, bwd

        xa.wait()
        out_ref[pl.ds(r * CH, CH), pl.ds(ring_lo, RCOLS)] = (
            p_ref[:, pl.ds(ring_lo, RCOLS)] + xrecv[:, pl.ds(ring_lo, RCOLS)]
        )
        steps = [make_step(0)]
        xb.wait()
        out_ref[pl.ds(r * CH, CH), pl.ds(mylack_lo, XCOLS)] = (
            p_ref[:, pl.ds(mylack_lo, XCOLS)] + xrecv[:, pl.ds(mylack_lo, XCOLS)]
        )
        xfwds = []
        for s in range(1, 4):
            fwd, bwd = steps[s - 1]
            fwd.wait()
            bwd.wait()
            steps.append(make_step(s))
            for o in ((r - s + N_RING) % N_RING, (r + s) % N_RING):
                xf = x_copy(o, len(xfwds))
                xf.start()
                xfwds.append(xf)
        fwd, bwd = steps[3]
        fwd.wait()
        bwd.wait()
        xf = x_copy((r + 4) % N_RING, len(xfwds))
        xf.start()
        xfwds.append(xf)
        for xf in xfwds:
            xf.wait()

    return pl.pallas_call(
        body,
        out_shape=jax.ShapeDtypeStruct((M, D), jnp.bfloat16),
        in_specs=[pl.BlockSpec(memory_space=pltpu.VMEM)],
        out_specs=pl.BlockSpec(memory_space=pltpu.VMEM),
        scratch_shapes=[
            pltpu.VMEM((CH, D), jnp.bfloat16),
            pltpu.SemaphoreType.DMA,
            pltpu.SemaphoreType.DMA,
            pltpu.SemaphoreType.DMA,
            pltpu.SemaphoreType.DMA,
            pltpu.SemaphoreType.DMA((4,)),
            pltpu.SemaphoreType.DMA((4,)),
            pltpu.SemaphoreType.DMA((4,)),
            pltpu.SemaphoreType.DMA((4,)),
            pltpu.SemaphoreType.DMA((7,)),
            pltpu.SemaphoreType.DMA((7,)),
        ],
        compiler_params=pltpu.CompilerParams(
            collective_id=0, vmem_limit_bytes=60 * 1024 * 1024
        ),
    )(partial)


# device time: 284239 ns/iter; 1.0111x vs baseline; 1.0111x over previous
import jax
import jax.numpy as jnp
from jax import lax
from jax.experimental import pallas as pl
from jax.experimental.pallas import tpu as pltpu

M = 4096
D = 4096
N_RING = 8
CH = M // N_RING
HCH = CH // 2
XCOLS = 1408
RCOLS = D - XCOLS


def _ring_yz(pos):
    y = jnp.where(pos < 4, 0, 1)
    z = jnp.where(pos < 4, pos, 7 - pos)
    return y, z


def kernel(dy, W):
    partial = _partial_gemm(dy, W)
    return _allreduce(partial).astype(jnp.float32)


def _partial_gemm(dy, W):
    F = dy.shape[1]
    WT = 256

    def body(dy_hbm, w_ref, out_ref, dyf, dyb, dy_sem):
        @pl.when(pl.program_id(0) == 0)
        def _():
            my_y = lax.axis_index("y")
            my_z = lax.axis_index("z")
            r = jnp.where(my_y == 0, my_z, 7 - my_z)
            cp = pltpu.make_async_copy(
                dy_hbm.at[pl.ds(r * CH, CH), :], dyf, dy_sem
            )
            cp.start()
            cp.wait()
            dyb[...] = dyf[...].astype(jnp.bfloat16)
        out_ref[...] = lax.dot_general(
            dyb[...],
            w_ref[...].astype(jnp.bfloat16),
            dimension_numbers=(((1,), (1,)), ((), ())),
            preferred_element_type=jnp.float32,
        ).astype(jnp.bfloat16)

    return pl.pallas_call(
        body,
        grid=(D // WT,),
        out_shape=jax.ShapeDtypeStruct((CH, D), jnp.bfloat16),
        in_specs=[
            pl.BlockSpec(memory_space=pl.ANY),
            pl.BlockSpec((WT, F), lambda i: (i, 0)),
        ],
        out_specs=pl.BlockSpec((CH, WT), lambda i: (0, i)),
        scratch_shapes=[
            pltpu.VMEM((CH, F), jnp.float32),
            pltpu.VMEM((CH, F), jnp.bfloat16),
            pltpu.SemaphoreType.DMA,
        ],
        compiler_params=pltpu.CompilerParams(
            vmem_limit_bytes=60 * 1024 * 1024
        ),
    )(dy, W)


def _allreduce(partial):
    def body(
        p_ref, out_ref, xrecv,
        xsa_sem, xra_sem, xsb_sem, xrb_sem,
        fs_sems, fr_sems, bs_sems, br_sems, xfs_sems, xfr_sems,
    ):
        my_x = lax.axis_index("x")
        my_y = lax.axis_index("y")
        my_z = lax.axis_index("z")
        r = jnp.where(my_y == 0, my_z, 7 - my_z)

        ry, rz = _ring_yz((r + 1) % N_RING)
        ly, lz = _ring_yz((r + 7) % N_RING)
        partner = (1 - my_x, my_y, my_z)
        right = (my_x, ry, rz)
        left = (my_x, ly, lz)

        ring_lo = jnp.where(my_x == 0, 0, XCOLS)
        xsend_lo = jnp.where(my_x == 0, 0, RCOLS)
        mylack_lo = jnp.where(my_x == 0, RCOLS, 0)
        partner_ring_lo = jnp.where(my_x == 0, XCOLS, 0)

        barrier_sem = pltpu.get_barrier_semaphore()
        for nbr in (partner, right, left):
            pl.semaphore_signal(
                barrier_sem, inc=1, device_id=nbr,
                device_id_type=pl.DeviceIdType.MESH,
            )
        pl.semaphore_wait(barrier_sem, 3)

        xa = pltpu.make_async_remote_copy(
            src_ref=p_ref.at[:, pl.ds(partner_ring_lo, RCOLS)],
            dst_ref=xrecv.at[:, pl.ds(partner_ring_lo, RCOLS)],
            send_sem=xsa_sem, recv_sem=xra_sem,
            device_id=partner, device_id_type=pl.DeviceIdType.MESH,
        )
        xb = pltpu.make_async_remote_copy(
            src_ref=p_ref.at[:, pl.ds(xsend_lo, XCOLS)],
            dst_ref=xrecv.at[:, pl.ds(xsend_lo, XCOLS)],
            send_sem=xsb_sem, recv_sem=xrb_sem,
            device_id=partner, device_id_type=pl.DeviceIdType.MESH,
        )
        xa.start()
        xb.start()

        def ring_copy(origin, rows_lo, rows, target, send_sem, recv_sem):
            ref = out_ref.at[pl.ds(origin * CH + rows_lo, rows),
                             pl.ds(ring_lo, RCOLS)]
            return pltpu.make_async_remote_copy(
                src_ref=ref, dst_ref=ref, send_sem=send_sem, recv_sem=recv_sem,
                device_id=target, device_id_type=pl.DeviceIdType.MESH,
            )

        def x_copy(origin, k):
            ref = out_ref.at[pl.ds(origin * CH, CH), pl.ds(xsend_lo, XCOLS)]
            return pltpu.make_async_remote_copy(
                src_ref=ref, dst_ref=ref,
                send_sem=xfs_sems.at[k], recv_sem=xfr_sems.at[k],
                device_id=partner, device_id_type=pl.DeviceIdType.MESH,
            )

        def make_step(s):
            if s < 3:
                fwd = ring_copy((r - s + N_RING) % N_RING, 0, CH, right,
                                fs_sems.at[s], fr_sems.at[s])
                bwd = ring_copy((r + s) % N_RING, 0, CH, left,
                                bs_sems.at[s], br_sems.at[s])
            else:
                fwd = ring_copy((r - 3 + N_RING) % N_RING, 0, HCH, right,
                                fs_sems.at[s], fr_sems.at[s])
                bwd = ring_copy((r + 3) % N_RING, HCH, HCH, left,
                                bs_sems.at[s], br_sems.at[s])
            fwd.start()
            bwd.start()
            return fwd, bwd

        xa.wait()
        out_ref[pl.ds(r * CH, CH), pl.ds(ring_lo, RCOLS)] = (
            p_ref[:, pl.ds(ring_lo, RCOLS)] + xrecv[:, pl.ds(ring_lo, RCOLS)]
        )
        steps = [make_step(0)]
        xb.wait()
        out_ref[pl.ds(r * CH, CH), pl.ds(mylack_lo, XCOLS)] = (
            p_ref[:, pl.ds(mylack_lo, XCOLS)] + xrecv[:, pl.ds(mylack_lo, XCOLS)]
        )
        xfwds = []
        for s in range(1, 4):
            fwd, bwd = steps[s - 1]
            fwd.wait()
            bwd.wait()
            steps.append(make_step(s))
            for o in ((r - s + N_RING) % N_RING, (r + s) % N_RING):
                xf = x_copy(o, len(xfwds))
                xf.start()
                xfwds.append(xf)
        fwd, bwd = steps[3]
        fwd.wait()
        bwd.wait()
        xf = x_copy((r + 4) % N_RING, len(xfwds))
        xf.start()
        xfwds.append(xf)
        for xf in xfwds:
            xf.wait()

    return pl.pallas_call(
        body,
        out_shape=jax.ShapeDtypeStruct((M, D), jnp.bfloat16),
        in_specs=[pl.BlockSpec(memory_space=pltpu.VMEM)],
        out_specs=pl.BlockSpec(memory_space=pltpu.VMEM),
        scratch_shapes=[
            pltpu.VMEM((CH, D), jnp.bfloat16),
            pltpu.SemaphoreType.DMA,
            pltpu.SemaphoreType.DMA,
            pltpu.SemaphoreType.DMA,
            pltpu.SemaphoreType.DMA,
            pltpu.SemaphoreType.DMA((4,)),
            pltpu.SemaphoreType.DMA((4,)),
            pltpu.SemaphoreType.DMA((4,)),
            pltpu.SemaphoreType.DMA((4,)),
            pltpu.SemaphoreType.DMA((7,)),
            pltpu.SemaphoreType.DMA((7,)),
        ],
        compiler_params=pltpu.CompilerParams(
            collective_id=0, vmem_limit_bytes=60 * 1024 * 1024
        ),
    )(partial)
